# Initial kernel scaffold; baseline (speedup 1.0000x reference)
import jax
import jax.numpy as jnp
from jax.experimental import pallas as pl


def kernel(x_branch, edge_index_branch, batch_branch, x_trunk, edge_index_trunk, batch_trunk, params):
    raise NotImplementedError("write your pallas kernel here")


# speedup vs baseline: 31.6540x; 31.6540x over previous
import functools

import jax
import jax.numpy as jnp
from jax import lax
from jax.experimental import pallas as pl
from jax.experimental.pallas import tpu as pltpu
from jax.experimental.pallas import tpu_sc as plsc

F32 = jnp.float32
EPS = 1e-16


def _dense1_body(x_ref, w_ref, aps_ref, apd_ref, h0, h1, h2, h3, as_ref, ad_ref):
    x = x_ref[...]
    w = w_ref[...]
    h = x[:, 0:1] * w[0:1, :] + x[:, 1:2] * w[1:2, :]
    h0[...] = h[:, 0:32]
    h1[...] = h[:, 32:64]
    h2[...] = h[:, 64:96]
    h3[...] = h[:, 96:128]
    as_ref[...] = jnp.dot(h, aps_ref[...], preferred_element_type=F32, precision=lax.Precision.HIGHEST)
    ad_ref[...] = jnp.dot(h, apd_ref[...], preferred_element_type=F32, precision=lax.Precision.HIGHEST)


def _dense1(x, w, aps, apd, bn=2000):
    n = x.shape[0]
    return pl.pallas_call(
        _dense1_body,
        grid=(n // bn,),
        in_specs=[
            pl.BlockSpec((bn, 2), lambda i: (i, 0)),
            pl.BlockSpec((2, 128), lambda i: (0, 0)),
            pl.BlockSpec((128, 2), lambda i: (0, 0)),
            pl.BlockSpec((128, 2), lambda i: (0, 0)),
        ],
        out_specs=[pl.BlockSpec((bn, 32), lambda i: (i, 0))] * 4
        + [pl.BlockSpec((bn, 2), lambda i: (i, 0))] * 2,
        out_shape=[jax.ShapeDtypeStruct((n, 32), F32)] * 4
        + [jax.ShapeDtypeStruct((n, 2), F32)] * 2,
    )(x, w, aps, apd)


def _elu(x):
    return jnp.where(x > 0, x, jnp.exp(x) - 1.0)


def _dense2_body(n0, n1, n2, n3, d0, d1, b1, w2, a2s, a2d, h0, h1, as_ref, ad_ref):
    x0 = jnp.concatenate([n0[...], n1[...]], axis=1) / (d0[...] + EPS)
    x1 = jnp.concatenate([n2[...], n3[...]], axis=1) / (d1[...] + EPS)
    x = jnp.concatenate([x0, x1], axis=1) + b1[...]
    x = _elu(x)
    h2 = jnp.dot(x, w2[...], preferred_element_type=F32, precision=lax.Precision.HIGHEST)
    h0[...] = h2[:, 0:32]
    h1[...] = h2[:, 32:64]
    as_ref[...] = jnp.dot(h2, a2s[...], preferred_element_type=F32, precision=lax.Precision.HIGHEST)
    ad_ref[...] = jnp.dot(h2, a2d[...], preferred_element_type=F32, precision=lax.Precision.HIGHEST)


def _dense2(n0, n1, n2, n3, d0, d1, b1, w2, a2s, a2d, bn=2000):
    n = n0.shape[0]
    sl = lambda i: (i, 0)
    z = lambda i: (0, 0)
    return pl.pallas_call(
        _dense2_body,
        grid=(n // bn,),
        in_specs=[pl.BlockSpec((bn, 32), sl)] * 4
        + [pl.BlockSpec((bn, 1), sl)] * 2
        + [
            pl.BlockSpec((1, 128), z),
            pl.BlockSpec((128, 64), z),
            pl.BlockSpec((64, 1), z),
            pl.BlockSpec((64, 1), z),
        ],
        out_specs=[pl.BlockSpec((bn, 32), sl)] * 2 + [pl.BlockSpec((bn, 1), sl)] * 2,
        out_shape=[jax.ShapeDtypeStruct((n, 32), F32)] * 2
        + [jax.ShapeDtypeStruct((n, 1), F32)] * 2,
    )(n0, n1, n2, n3, d0, d1, b1, w2, a2s, a2d)


def _pool_mlp_body(num0, num1, den, b2, batch, w1m, b1m, w2m, b2m, out, acc, cnt):
    i = pl.program_id(0)
    ng = pl.num_programs(0)

    @pl.when(i == 0)
    def _():
        acc[...] = jnp.zeros_like(acc)
        cnt[...] = jnp.zeros_like(cnt)

    x = jnp.concatenate([num0[...], num1[...]], axis=1) / (den[...] + EPS) + b2[...]
    x = _elu(x)
    bn = x.shape[0]
    oh = (batch[...] == lax.broadcasted_iota(jnp.int32, (bn, 128), 1)).astype(F32)
    acc[...] += lax.dot_general(
        oh, x, (((0,), (0,)), ((), ())), preferred_element_type=F32, precision=lax.Precision.HIGHEST
    )
    cnt[...] += lax.dot_general(
        oh, jnp.ones((bn, 1), F32), (((0,), (0,)), ((), ())), preferred_element_type=F32, precision=lax.Precision.HIGHEST
    )

    @pl.when(i == ng - 1)
    def _():
        xg = acc[...] / jnp.maximum(cnt[...], 1.0)
        hmid = jnp.maximum(jnp.dot(xg, w1m[...], preferred_element_type=F32, precision=lax.Precision.HIGHEST) + b1m[...], 0.0)
        out[...] = jnp.dot(hmid, w2m[...], preferred_element_type=F32, precision=lax.Precision.HIGHEST) + b2m[...]


def _pool_mlp(num0, num1, den, b2, batch, w1m, b1m, w2m, b2m, bn=2000):
    n = num0.shape[0]
    sl = lambda i: (i, 0)
    z = lambda i: (0, 0)
    return pl.pallas_call(
        _pool_mlp_body,
        grid=(n // bn,),
        in_specs=[pl.BlockSpec((bn, 32), sl)] * 2
        + [
            pl.BlockSpec((bn, 1), sl),
            pl.BlockSpec((1, 64), z),
            pl.BlockSpec((bn, 1), sl),
            pl.BlockSpec((64, 64), z),
            pl.BlockSpec((1, 64), z),
            pl.BlockSpec((64, 64), z),
            pl.BlockSpec((1, 64), z),
        ],
        out_specs=pl.BlockSpec((128, 64), z),
        out_shape=jax.ShapeDtypeStruct((128, 64), F32),
        scratch_shapes=[pltpu.VMEM((128, 64), F32), pltpu.VMEM((128, 1), F32)],
    )(num0, num1, den, b2, batch, w1m, b1m, w2m, b2m)


def _combine_body(bo, to, w1, b1, w2, b2, out):
    c = bo[...] * to[...]
    h = jnp.maximum(jnp.dot(c, w1[...], preferred_element_type=F32, precision=lax.Precision.HIGHEST) + b1[...], 0.0)
    out[...] = jnp.tanh(jnp.dot(h, w2[...], preferred_element_type=F32, precision=lax.Precision.HIGHEST) + b2[...])


def _combine(bo, to, w1, b1, w2, b2):
    return pl.pallas_call(
        _combine_body,
        out_shape=jax.ShapeDtypeStruct((bo.shape[0], 2), F32),
    )(bo, to, w1, b1, w2, b2)


_B = 1024
_JROWS = _B // 128
_HB = 512
_ZCH = 400
_EALIGN = 16 * _B


def _make_edge_kernel(n, e_pad, heads, nslabs):
    spc = nslabs // 2
    ept = e_pad // 16
    nch = ept // _B
    nzch = n // _ZCH
    acc_rows = n + 8

    mesh = plsc.VectorSubcoreMesh(core_axis_name="c", subcore_axis_name="s")

    def body(*refs):
        (src_r, dst_r), rest = refs[:2], refs[2:]
        as_t = rest[:heads]; rest = rest[heads:]
        ad_t = rest[:heads]; rest = rest[heads:]
        h_t = rest[:nslabs]; rest = rest[nslabs:]
        num_o = rest[:nslabs]; rest = rest[nslabs:]
        den_o = rest[:heads]; rest = rest[heads:]
        idx_s, idx_d, asg, adg, eebuf, hg, acc, den_acc = rest

        c = lax.axis_index("c")
        s = lax.axis_index("s")
        ebase = s * ept
        rbase = s * (ept // 128)

        def my_zchunks(fn):
            for j in range((nzch + 15) // 16):
                k = s + 16 * j
                @pl.when(k < nzch)
                def _():
                    fn(pl.multiple_of(k * _ZCH, 8))

        zero16 = jnp.zeros((16,), F32)

        def core_work(core, my_h, my_num, my_as, my_ad, my_den, do_den):
            for p in range(spc):
                h_r = my_h[p]
                num_r = my_num[p]

                def zf_body(b, _):
                    hg[b, 0:16] = zero16
                    hg[b, 16:32] = zero16
                    return ()

                lax.fori_loop(0, _ZCH, zf_body, ())

                def zf1_body(g, _):
                    asg[pl.ds(pl.multiple_of(g * 16, 8), 16)] = zero16
                    return ()

                lax.fori_loop(0, _ZCH // 16 + 1, zf1_body, ())

                def zfn(r0, first=(p == 0)):
                    pltpu.sync_copy(hg.at[pl.ds(0, _ZCH)], acc.at[pl.ds(r0, _ZCH)])
                    if first and do_den:
                        pltpu.sync_copy(asg.at[pl.ds(0, _ZCH)], den_acc.at[pl.ds(r0, _ZCH)])
                my_zchunks(zfn)
                plsc.subcore_barrier()

                def chunk_body(i, _, first=(p == 0)):
                    roff = pl.multiple_of(rbase + i * _JROWS, 8)
                    pltpu.sync_copy(src_r.at[pl.ds(roff, _JROWS)], idx_s)
                    pltpu.sync_copy(dst_r.at[pl.ds(roff, _JROWS)], idx_d)
                    for j in range(_JROWS):
                        pltpu.sync_copy(
                            my_as.at[idx_s.at[j]],
                            asg.at[pl.ds(j * 128, 128)],
                        )
                        pltpu.sync_copy(
                            my_ad.at[idx_d.at[j]],
                            adg.at[pl.ds(j * 128, 128)],
                        )

                    def ee_body(g, _):
                        o16 = pl.multiple_of(g * 16, 8)
                        t = asg[pl.ds(o16, 16)] + adg[pl.ds(o16, 16)]
                        e = jnp.maximum(t, t * 0.2)
                        eebuf[pl.ds(o16, 16)] = jnp.exp(e)
                        return ()

                    lax.fori_loop(0, _B // 16, ee_body, ())
                    if first and do_den:
                        for j in range(_JROWS):
                            pltpu.sync_copy(
                                eebuf.at[pl.ds(j * 128, 128)],
                                den_acc.at[idx_d.at[j]],
                                add=True,
                            )
                    for half in range(_B // _HB):
                        jo = half * (_HB // 128)
                        for j in range(_HB // 128):
                            pltpu.sync_copy(
                                h_r.at[idx_s.at[jo + j]],
                                hg.at[pl.ds(j * 128, 128)],
                            )

                        def sc_body(g, _, half=half):
                            o16 = pl.multiple_of(g * 16, 8)
                            ee16 = eebuf[pl.ds(half * _HB + o16, 16)]
                            for l in range(16):
                                b = g * 16 + l
                                see = ee16[l]
                                hg[b, 0:16] = hg[b, 0:16] * see
                                hg[b, 16:32] = hg[b, 16:32] * see
                            return ()

                        lax.fori_loop(0, _HB // 16, sc_body, ())

                        for j in range(_HB // 128):
                            pltpu.sync_copy(
                                hg.at[pl.ds(j * 128, 128)],
                                acc.at[idx_d.at[jo + j]],
                                add=True,
                            )
                    return ()

                lax.fori_loop(0, nch, chunk_body, ())
                plsc.subcore_barrier()

                def dfn(r0, first=(p == 0)):
                    pltpu.sync_copy(acc.at[pl.ds(r0, _ZCH)], hg.at[pl.ds(0, _ZCH)])
                    pltpu.sync_copy(hg.at[pl.ds(0, _ZCH)], num_r.at[pl.ds(r0, _ZCH)])
                    if first and do_den:
                        pltpu.sync_copy(den_acc.at[pl.ds(r0, _ZCH)], asg.at[pl.ds(0, _ZCH)])
                        pltpu.sync_copy(asg.at[pl.ds(0, _ZCH)], my_den.at[pl.ds(r0, _ZCH)])
                my_zchunks(dfn)
                plsc.subcore_barrier()

        for core in range(2):
            hd = core if heads == 2 else 0

            @pl.when(c == core)
            def _(core=core, hd=hd):
                core_work(
                    core,
                    [h_t[core * spc + p] for p in range(spc)],
                    [num_o[core * spc + p] for p in range(spc)],
                    as_t[hd],
                    ad_t[hd],
                    den_o[hd] if core < heads else den_o[0],
                    core < heads,
                )

    kern = pl.kernel(
        body,
        out_type=[jax.ShapeDtypeStruct((n, 32), F32)] * nslabs
        + [jax.ShapeDtypeStruct((n,), F32)] * heads,
        mesh=mesh,
        compiler_params=pltpu.CompilerParams(use_tc_tiling_on_sc=False),
        scratch_types=[
            pltpu.VMEM((_JROWS, 128), jnp.int32),
            pltpu.VMEM((_JROWS, 128), jnp.int32),
            pltpu.VMEM((_B,), F32),
            pltpu.VMEM((_B,), F32),
            pltpu.VMEM((_B,), F32),
            pltpu.VMEM((_HB, 32), F32),
            pltpu.VMEM_SHARED((acc_rows, 32), F32),
            pltpu.VMEM_SHARED((acc_rows,), F32),
        ],
    )
    return kern


def _branch_net(x, edge_index, batch, p, pre, edge_k1, edge_k2):
    n = x.shape[0]
    e = edge_index.shape[1]
    e_pad = ((e + _EALIGN - 1) // _EALIGN) * _EALIGN

    src = edge_index[0]
    dst = edge_index[1]
    src = jnp.concatenate([src, jnp.zeros((e_pad - e,), jnp.int32)])
    dst = jnp.concatenate([dst, jnp.full((e_pad - e,), n, jnp.int32)])
    src2d = src.reshape(e_pad // 128, 128)
    dst2d = dst.reshape(e_pad // 128, 128)

    asw = p[pre + "gat1_as"]
    adw = p[pre + "gat1_ad"]
    z64 = jnp.zeros((64,), F32)
    aps = jnp.stack(
        [jnp.concatenate([asw[0], z64]), jnp.concatenate([z64, asw[1]])], axis=1
    )
    apd = jnp.stack(
        [jnp.concatenate([adw[0], z64]), jnp.concatenate([z64, adw[1]])], axis=1
    )
    h0, h1, h2, h3, al_s, al_d = _dense1(x, p[pre + "gat1_W"], aps, apd)
    as0 = jnp.asarray(al_s[:, 0])
    as1 = jnp.asarray(al_s[:, 1])
    ad0 = jnp.asarray(al_d[:, 0])
    ad1 = jnp.asarray(al_d[:, 1])

    n0, n1, n2, n3, d0, d1 = edge_k1(
        src2d, dst2d, as0, as1, ad0, ad1, h0, h1, h2, h3
    )

    b1 = p[pre + "gat1_b"].reshape(1, 128)
    g0, g1, a2s, a2d = _dense2(
        n0, n1, n2, n3,
        d0.reshape(n, 1), d1.reshape(n, 1),
        b1, p[pre + "gat2_W"],
        p[pre + "gat2_as"].reshape(64, 1), p[pre + "gat2_ad"].reshape(64, 1),
    )

    m0, m1, dd = edge_k2(
        src2d, dst2d,
        a2s.reshape(n), a2d.reshape(n),
        g0, g1,
    )

    return _pool_mlp(
        m0, m1, dd.reshape(n, 1),
        p[pre + "gat2_b"].reshape(1, 64),
        batch.reshape(n, 1),
        p[pre + "mlp_W1"], p[pre + "mlp_b1"].reshape(1, 64),
        p[pre + "mlp_W2"], p[pre + "mlp_b2"].reshape(1, 64),
    )


def kernel(x_branch, edge_index_branch, batch_branch, x_trunk, edge_index_trunk,
           batch_trunk, params):
    n = x_branch.shape[0]
    e = edge_index_branch.shape[1]
    e_pad = ((e + _EALIGN - 1) // _EALIGN) * _EALIGN
    edge_k1 = _make_edge_kernel(n, e_pad, heads=2, nslabs=4)
    edge_k2 = _make_edge_kernel(n, e_pad, heads=1, nslabs=2)

    bo = _branch_net(x_branch, edge_index_branch, batch_branch, params, "b_",
                     edge_k1, edge_k2)
    to = _branch_net(x_trunk, edge_index_trunk, batch_trunk, params, "t_",
                     edge_k1, edge_k2)

    h = _combine(
        bo, to,
        params["f_W1"], params["f_b1"].reshape(1, 64),
        params["f_W2"], params["f_b2"].reshape(1, 2),
    )
    return h
